# baseline (speedup 1.0000x reference)
import functools

import jax
import jax.numpy as jnp
from jax import lax
from jax.experimental import pallas as pl
from jax.experimental.pallas import tpu as pltpu
from jax.experimental.pallas import tpu_sc as plsc

N_NODES = 10000
N_PAD = 10240
N_EDGES = 320000
D = 128
HALF = D // 2

NC = 2
NS = 16
CHUNK = 128
K_BLK = 4
E_PER_T = 20480
E_PAD = NS * E_PER_T
N_CHUNKS = E_PER_T // CHUNK
N_BLOCKS = N_CHUNKS // K_BLK
CNT_SPLIT = N_BLOCKS // 2
ROWS_PER_TILE = N_PAD // NS
N_ZSTRIPES = ROWS_PER_TILE // CHUNK
CNT_W = 16


def _sc_body(h2_hbm, src_hbm, dst_hbm, ones_hbm,
             acc_out, cnt_out,
             src_v, dst_v, rows_v, ones_v, zcnt_v, acc_sh, cnt_sh,
             sem_g, sem_s, sem_s2, sem_c, sem_i):
    c = lax.axis_index("c")
    s = lax.axis_index("s")
    r0 = s * ROWS_PER_TILE

    z16 = jnp.zeros((16,), jnp.float32)

    def zbuf(i, carry):
        for j in range(HALF // 16):
            rows_v[0, 0, i, pl.ds(j * 16, 16)] = z16
        zcnt_v[i, :] = z16
        return carry

    lax.fori_loop(0, CHUNK, zbuf, 0)

    def zstripe(k, carry):
        pltpu.sync_copy(rows_v.at[0, 0],
                        acc_sh.at[pl.ds(r0 + k * CHUNK, CHUNK)])
        pltpu.sync_copy(zcnt_v, cnt_sh.at[pl.ds(r0 + k * CHUNK, CHUNK)])
        return carry

    lax.fori_loop(0, N_ZSTRIPES, zstripe, 0)

    pltpu.sync_copy(ones_hbm, ones_v)
    blk0 = s * N_BLOCKS
    pltpu.sync_copy(src_hbm.at[blk0], src_v.at[0])
    pltpu.sync_copy(dst_hbm.at[blk0], dst_v.at[0])
    pltpu.async_copy(src_hbm.at[blk0 + 1], src_v.at[1], sem_i)
    pltpu.async_copy(dst_hbm.at[blk0 + 1], dst_v.at[1], sem_i)
    plsc.subcore_barrier()

    def fire_gathers(rset, iset):
        for b in range(K_BLK):
            pltpu.async_copy(h2_hbm.at[src_v.at[iset, b]],
                             rows_v.at[rset, b], sem_g)

    def xform_idx(iset):
        for k in range(K_BLK):
            for j in range(CHUNK // 16):
                sl = pl.ds(j * 16, 16)
                src_v[iset, k, sl] = src_v[iset, k, sl] * 2 + c

    xform_idx(0)
    fire_gathers(0, 0)

    def block(o, carry):
        r = lax.rem(o, 2)
        i3 = lax.rem(o, 3)
        for b in range(K_BLK):
            pltpu.make_async_copy(h2_hbm.at[src_v.at[i3, b]],
                                  rows_v.at[r, b], sem_g).wait()
        @pl.when(r == 0)
        def _():
            for b in range(K_BLK):
                pltpu.async_copy(rows_v.at[r, b], acc_sh.at[dst_v.at[i3, b]],
                                 sem_s, add=True)

        @pl.when(r == 1)
        def _():
            for b in range(K_BLK):
                pltpu.async_copy(rows_v.at[r, b], acc_sh.at[dst_v.at[i3, b]],
                                 sem_s2, add=True)

        @pl.when(jnp.logical_xor(o < CNT_SPLIT, c == 1))
        def _():
            oc = [pltpu.async_copy(ones_v, cnt_sh.at[dst_v.at[i3, b]],
                                   sem_c, add=True)
                  for b in range(K_BLK)]
            for d in oc:
                d.wait()

        @pl.when(jnp.logical_and(o >= 1, r == 1))
        def _():
            for b in range(K_BLK):
                pltpu.make_async_copy(rows_v.at[0, b],
                                      acc_sh.at[dst_v.at[i3, b]],
                                      sem_s).wait()

        @pl.when(jnp.logical_and(o >= 1, r == 0))
        def _():
            for b in range(K_BLK):
                pltpu.make_async_copy(rows_v.at[1, b],
                                      acc_sh.at[dst_v.at[i3, b]],
                                      sem_s2).wait()

        @pl.when(o + 1 < N_BLOCKS)
        def _():
            i3n = lax.rem(o + 1, 3)
            pltpu.make_async_copy(src_hbm.at[blk0 + o + 1],
                                  src_v.at[i3n], sem_i).wait()
            pltpu.make_async_copy(dst_hbm.at[blk0 + o + 1],
                                  dst_v.at[i3n], sem_i).wait()
            xform_idx(i3n)

        @pl.when(o + 2 < N_BLOCKS)
        def _():
            i3nn = lax.rem(o + 2, 3)
            pltpu.async_copy(src_hbm.at[blk0 + o + 2], src_v.at[i3nn], sem_i)
            pltpu.async_copy(dst_hbm.at[blk0 + o + 2], dst_v.at[i3nn], sem_i)

        @pl.when(o + 1 < N_BLOCKS)
        def _():
            fire_gathers(1 - r, lax.rem(o + 1, 3))
        return carry

    lax.fori_loop(0, N_BLOCKS, block, 0)

    _last_r = (N_BLOCKS - 1) % 2
    _last_sem = sem_s2 if _last_r == 1 else sem_s
    for b in range(K_BLK):
        pltpu.make_async_copy(rows_v.at[_last_r, b],
                              acc_sh.at[dst_v.at[0, b]], _last_sem).wait()
    plsc.subcore_barrier()

    pltpu.sync_copy(acc_sh.at[pl.ds(r0, ROWS_PER_TILE)],
                    acc_out.at[pl.ds(r0, ROWS_PER_TILE),
                               pl.ds(c * HALF, HALF)])
    pltpu.sync_copy(cnt_sh.at[pl.ds(r0, ROWS_PER_TILE)],
                    cnt_out.at[pl.ds(r0, ROWS_PER_TILE),
                               pl.ds(c * CNT_W, CNT_W)])


_sc_gather_scatter = functools.partial(
    pl.kernel,
    out_type=(
        jax.ShapeDtypeStruct((N_PAD, D), jnp.float32),
        jax.ShapeDtypeStruct((N_PAD, 2 * CNT_W), jnp.float32),
    ),
    mesh=plsc.VectorSubcoreMesh(core_axis_name="c", subcore_axis_name="s"),
    compiler_params=pltpu.CompilerParams(use_tc_tiling_on_sc=False),
    scratch_types=[
        pltpu.VMEM((3, K_BLK, CHUNK), jnp.int32),
        pltpu.VMEM((3, K_BLK, CHUNK), jnp.int32),
        pltpu.VMEM((2, K_BLK, CHUNK, HALF), jnp.float32),
        pltpu.VMEM((CHUNK, CNT_W), jnp.float32),
        pltpu.VMEM((CHUNK, CNT_W), jnp.float32),
        pltpu.VMEM_SHARED((N_PAD, HALF), jnp.float32),
        pltpu.VMEM_SHARED((N_PAD, CNT_W), jnp.float32),
        pltpu.SemaphoreType.DMA,
        pltpu.SemaphoreType.DMA,
        pltpu.SemaphoreType.DMA,
        pltpu.SemaphoreType.DMA,
        pltpu.SemaphoreType.DMA,
    ],
)(_sc_body)


def _dense_body(acc_ref, cnt_ref, h_ref, wn_ref, ws_ref, b_ref,
                gamma_ref, beta_ref, out_ref):
    summed = acc_ref[:N_NODES, :]
    count = cnt_ref[:N_NODES, 0:1] + cnt_ref[:N_NODES, CNT_W:CNT_W + 1]
    count = jnp.maximum(count, 1.0)
    mean_agg = summed / count
    out = (jnp.dot(mean_agg, wn_ref[...], preferred_element_type=jnp.float32)
           + jnp.dot(h_ref[...], ws_ref[...], preferred_element_type=jnp.float32)
           + b_ref[...])
    out = jnp.maximum(out, 0.0)
    mu = jnp.mean(out, axis=0, keepdims=True)
    var = jnp.mean((out - mu) ** 2, axis=0, keepdims=True)
    out = (out - mu) * lax.rsqrt(var + 1e-5)
    out_ref[...] = out * gamma_ref[...] + beta_ref[...]


_dense = pl.pallas_call(
    _dense_body,
    out_shape=jax.ShapeDtypeStruct((N_NODES, D), jnp.float32),
)


def kernel(feature, edge_index, W_neigh, W_self, b, gamma, beta):
    npad = E_PAD - N_EDGES
    src = jnp.concatenate(
        [edge_index[0].astype(jnp.int32), jnp.zeros((npad,), jnp.int32)]
    ).reshape(NS * N_BLOCKS, K_BLK, CHUNK)
    dst = jnp.concatenate(
        [edge_index[1].astype(jnp.int32), jnp.full((npad,), N_NODES, jnp.int32)]
    ).reshape(NS * N_BLOCKS, K_BLK, CHUNK)
    h2 = feature.reshape(2 * N_NODES, HALF)
    ones = jnp.ones((CHUNK, CNT_W), jnp.float32)
    acc, cnt = _sc_gather_scatter(h2, src, dst, ones)
    return _dense(acc, cnt, feature, W_neigh.T, W_self.T,
                  b.reshape(1, D), gamma.reshape(1, D), beta.reshape(1, D))


# speedup vs baseline: 2.5575x; 2.5575x over previous
import functools

import jax
import jax.numpy as jnp
from jax import lax
from jax.experimental import pallas as pl
from jax.experimental.pallas import tpu as pltpu
from jax.experimental.pallas import tpu_sc as plsc

N_NODES = 10000
N_PAD = 10240
N_EDGES = 320000
D = 128
HALF = D // 2

NC = 2
NS = 16
E_PER_T = N_EDGES // NS
CHUNK = 80
N_CHUNKS = E_PER_T // CHUNK
K_BLK = 5
N_BLOCKS = N_CHUNKS // K_BLK
CNT_SPLIT = N_BLOCKS // 2
ROWS_PER_TILE = N_PAD // NS
N_ZSTRIPES = ROWS_PER_TILE // CHUNK
CNT_W = 16


def _sc_body(h2_hbm, src_hbm, dst_hbm, ones_hbm,
             acc_out, cnt_out,
             src_v, dst_v, rows_v, ones_v, zcnt_v, acc_sh, cnt_sh,
             sem_g, sem_s, sem_s2, sem_c, sem_i):
    c = lax.axis_index("c")
    s = lax.axis_index("s")
    r0 = s * ROWS_PER_TILE

    z16 = jnp.zeros((16,), jnp.float32)

    def zbuf(i, carry):
        for j in range(HALF // 16):
            rows_v[0, 0, i, pl.ds(j * 16, 16)] = z16
        zcnt_v[i, :] = z16
        return carry

    lax.fori_loop(0, CHUNK, zbuf, 0)

    def zstripe(k, carry):
        pltpu.sync_copy(rows_v.at[0, 0],
                        acc_sh.at[pl.ds(r0 + k * CHUNK, CHUNK)])
        pltpu.sync_copy(zcnt_v, cnt_sh.at[pl.ds(r0 + k * CHUNK, CHUNK)])
        return carry

    lax.fori_loop(0, N_ZSTRIPES, zstripe, 0)

    pltpu.sync_copy(ones_hbm, ones_v)
    blk0 = s * N_BLOCKS
    pltpu.sync_copy(src_hbm.at[blk0], src_v.at[0])
    pltpu.sync_copy(dst_hbm.at[blk0], dst_v.at[0])
    pltpu.async_copy(src_hbm.at[blk0 + 1], src_v.at[1], sem_i)
    pltpu.async_copy(dst_hbm.at[blk0 + 1], dst_v.at[1], sem_i)
    plsc.subcore_barrier()

    def fire_gathers(rset, iset):
        for b in range(K_BLK):
            pltpu.async_copy(h2_hbm.at[src_v.at[iset, b]],
                             rows_v.at[rset, b], sem_g)

    def xform_idx(iset):
        for k in range(K_BLK):
            for j in range(CHUNK // 16):
                sl = pl.ds(j * 16, 16)
                src_v[iset, k, sl] = src_v[iset, k, sl] * 2 + c

    xform_idx(0)
    fire_gathers(0, 0)

    def block(o, carry):
        r = lax.rem(o, 2)
        i3 = lax.rem(o, 3)
        for b in range(K_BLK):
            pltpu.make_async_copy(h2_hbm.at[src_v.at[i3, b]],
                                  rows_v.at[r, b], sem_g).wait()
        @pl.when(r == 0)
        def _():
            for b in range(K_BLK):
                pltpu.async_copy(rows_v.at[r, b], acc_sh.at[dst_v.at[i3, b]],
                                 sem_s, add=True)

        @pl.when(r == 1)
        def _():
            for b in range(K_BLK):
                pltpu.async_copy(rows_v.at[r, b], acc_sh.at[dst_v.at[i3, b]],
                                 sem_s2, add=True)

        @pl.when(jnp.logical_xor(o < CNT_SPLIT, c == 1))
        def _():
            oc = [pltpu.async_copy(ones_v, cnt_sh.at[dst_v.at[i3, b]],
                                   sem_c, add=True)
                  for b in range(K_BLK)]
            for d in oc:
                d.wait()

        @pl.when(jnp.logical_and(o >= 1, r == 1))
        def _():
            for b in range(K_BLK):
                pltpu.make_async_copy(rows_v.at[0, b],
                                      acc_sh.at[dst_v.at[i3, b]],
                                      sem_s).wait()

        @pl.when(jnp.logical_and(o >= 1, r == 0))
        def _():
            for b in range(K_BLK):
                pltpu.make_async_copy(rows_v.at[1, b],
                                      acc_sh.at[dst_v.at[i3, b]],
                                      sem_s2).wait()

        @pl.when(o + 1 < N_BLOCKS)
        def _():
            i3n = lax.rem(o + 1, 3)
            pltpu.make_async_copy(src_hbm.at[blk0 + o + 1],
                                  src_v.at[i3n], sem_i).wait()
            pltpu.make_async_copy(dst_hbm.at[blk0 + o + 1],
                                  dst_v.at[i3n], sem_i).wait()
            xform_idx(i3n)

        @pl.when(o + 2 < N_BLOCKS)
        def _():
            i3nn = lax.rem(o + 2, 3)
            pltpu.async_copy(src_hbm.at[blk0 + o + 2], src_v.at[i3nn], sem_i)
            pltpu.async_copy(dst_hbm.at[blk0 + o + 2], dst_v.at[i3nn], sem_i)

        @pl.when(o + 1 < N_BLOCKS)
        def _():
            fire_gathers(1 - r, lax.rem(o + 1, 3))
        return carry

    lax.fori_loop(0, N_BLOCKS, block, 0)

    _last_r = (N_BLOCKS - 1) % 2
    _last_sem = sem_s2 if _last_r == 1 else sem_s
    for b in range(K_BLK):
        pltpu.make_async_copy(rows_v.at[_last_r, b],
                              acc_sh.at[dst_v.at[0, b]], _last_sem).wait()
    plsc.subcore_barrier()

    pltpu.sync_copy(acc_sh.at[pl.ds(r0, ROWS_PER_TILE)],
                    acc_out.at[pl.ds(r0, ROWS_PER_TILE),
                               pl.ds(c * HALF, HALF)])
    pltpu.sync_copy(cnt_sh.at[pl.ds(r0, ROWS_PER_TILE)],
                    cnt_out.at[pl.ds(r0, ROWS_PER_TILE),
                               pl.ds(c * CNT_W, CNT_W)])


_sc_gather_scatter = functools.partial(
    pl.kernel,
    out_type=(
        jax.ShapeDtypeStruct((N_PAD, D), jnp.float32),
        jax.ShapeDtypeStruct((N_PAD, 2 * CNT_W), jnp.float32),
    ),
    mesh=plsc.VectorSubcoreMesh(core_axis_name="c", subcore_axis_name="s"),
    compiler_params=pltpu.CompilerParams(use_tc_tiling_on_sc=False),
    scratch_types=[
        pltpu.VMEM((3, K_BLK, CHUNK), jnp.int32),
        pltpu.VMEM((3, K_BLK, CHUNK), jnp.int32),
        pltpu.VMEM((2, K_BLK, CHUNK, HALF), jnp.float32),
        pltpu.VMEM((CHUNK, CNT_W), jnp.float32),
        pltpu.VMEM((CHUNK, CNT_W), jnp.float32),
        pltpu.VMEM_SHARED((N_PAD, HALF), jnp.float32),
        pltpu.VMEM_SHARED((N_PAD, CNT_W), jnp.float32),
        pltpu.SemaphoreType.DMA,
        pltpu.SemaphoreType.DMA,
        pltpu.SemaphoreType.DMA,
        pltpu.SemaphoreType.DMA,
        pltpu.SemaphoreType.DMA,
    ],
)(_sc_body)


def _dense_body(acc_ref, cnt_ref, h_ref, wn_ref, ws_ref, b_ref,
                gamma_ref, beta_ref, out_ref):
    summed = acc_ref[:N_NODES, :]
    count = cnt_ref[:N_NODES, 0:1] + cnt_ref[:N_NODES, CNT_W:CNT_W + 1]
    count = jnp.maximum(count, 1.0)
    mean_agg = summed / count
    out = (jnp.dot(mean_agg, wn_ref[...], preferred_element_type=jnp.float32)
           + jnp.dot(h_ref[...], ws_ref[...], preferred_element_type=jnp.float32)
           + b_ref[...])
    out = jnp.maximum(out, 0.0)
    mu = jnp.mean(out, axis=0, keepdims=True)
    var = jnp.mean((out - mu) ** 2, axis=0, keepdims=True)
    out = (out - mu) * lax.rsqrt(var + 1e-5)
    out_ref[...] = out * gamma_ref[...] + beta_ref[...]


_dense = pl.pallas_call(
    _dense_body,
    out_shape=jax.ShapeDtypeStruct((N_NODES, D), jnp.float32),
)


def kernel(feature, edge_index, W_neigh, W_self, b, gamma, beta):
    src = edge_index[0].astype(jnp.int32).reshape(NS * N_BLOCKS, K_BLK, CHUNK)
    dst = edge_index[1].astype(jnp.int32).reshape(NS * N_BLOCKS, K_BLK, CHUNK)
    h2 = feature.reshape(2 * N_NODES, HALF)
    ones = jnp.ones((CHUNK, CNT_W), jnp.float32)
    acc, cnt = _sc_gather_scatter(h2, src, dst, ones)
    return _dense(acc, cnt, feature, W_neigh.T, W_self.T,
                  b.reshape(1, D), gamma.reshape(1, D), beta.reshape(1, D))


# speedup vs baseline: 2.6051x; 1.0186x over previous
import functools

import jax
import jax.numpy as jnp
from jax import lax
from jax.experimental import pallas as pl
from jax.experimental.pallas import tpu as pltpu
from jax.experimental.pallas import tpu_sc as plsc

N_NODES = 10000
N_PAD = 10240
N_EDGES = 320000
D = 128
HALF = D // 2

NC = 2
NS = 16
E_PER_T = N_EDGES // NS
CHUNK = 80
N_CHUNKS = E_PER_T // CHUNK
K_BLK = 5
N_BLOCKS = N_CHUNKS // K_BLK
CNT_SPLIT = N_BLOCKS // 2
ROWS_PER_TILE = N_PAD // NS
N_ZSTRIPES = ROWS_PER_TILE // CHUNK
CNT_W = 16


def _sc_body(h2_hbm, src_hbm, dst_hbm, ones_hbm,
             acc_out, cnt_out,
             src_v, dst_v, rows_v, ones_v, zcnt_v, acc_sh, cnt_sh,
             sem_g, sem_s, sem_s2, sem_c, sem_i):
    c = lax.axis_index("c")
    s = lax.axis_index("s")
    r0 = s * ROWS_PER_TILE

    z16 = jnp.zeros((16,), jnp.float32)

    def zbuf(i, carry):
        for j in range(HALF // 16):
            rows_v[0, 0, i, pl.ds(j * 16, 16)] = z16
        zcnt_v[i, :] = z16
        return carry

    lax.fori_loop(0, CHUNK, zbuf, 0)

    blk0 = s * N_BLOCKS
    pltpu.async_copy(src_hbm.at[blk0], src_v.at[0], sem_g)
    pltpu.async_copy(dst_hbm.at[blk0], dst_v.at[0], sem_g)
    pltpu.async_copy(ones_hbm, ones_v, sem_g)

    def zstripe(k, carry):
        pltpu.async_copy(rows_v.at[0, 0],
                         acc_sh.at[pl.ds(r0 + k * CHUNK, CHUNK)], sem_s)
        pltpu.async_copy(zcnt_v,
                         cnt_sh.at[pl.ds(r0 + k * CHUNK, CHUNK)], sem_s2)
        return carry

    lax.fori_loop(0, N_ZSTRIPES, zstripe, 0)
    pltpu.async_copy(src_hbm.at[blk0 + 1], src_v.at[1], sem_i)
    pltpu.async_copy(dst_hbm.at[blk0 + 1], dst_v.at[1], sem_i)

    def zdrain(k, carry):
        pltpu.make_async_copy(rows_v.at[0, 0],
                              acc_sh.at[pl.ds(r0, CHUNK)], sem_s).wait()
        pltpu.make_async_copy(zcnt_v,
                              cnt_sh.at[pl.ds(r0, CHUNK)], sem_s2).wait()
        return carry

    lax.fori_loop(0, N_ZSTRIPES, zdrain, 0)
    pltpu.make_async_copy(src_hbm.at[blk0], src_v.at[0], sem_g).wait()
    pltpu.make_async_copy(dst_hbm.at[blk0], dst_v.at[0], sem_g).wait()
    pltpu.make_async_copy(ones_hbm, ones_v, sem_g).wait()
    plsc.subcore_barrier()

    def fire_gathers(rset, iset):
        for b in range(K_BLK):
            pltpu.async_copy(h2_hbm.at[src_v.at[iset, b]],
                             rows_v.at[rset, b], sem_g)

    def xform_idx(iset):
        for k in range(K_BLK):
            for j in range(CHUNK // 16):
                sl = pl.ds(j * 16, 16)
                src_v[iset, k, sl] = src_v[iset, k, sl] * 2 + c

    xform_idx(0)
    fire_gathers(0, 0)

    def block(o, carry):
        r = lax.rem(o, 2)
        i3 = lax.rem(o, 3)
        for b in range(K_BLK):
            pltpu.make_async_copy(h2_hbm.at[src_v.at[i3, b]],
                                  rows_v.at[r, b], sem_g).wait()
        @pl.when(r == 0)
        def _():
            for b in range(K_BLK):
                pltpu.async_copy(rows_v.at[r, b], acc_sh.at[dst_v.at[i3, b]],
                                 sem_s, add=True)

        @pl.when(r == 1)
        def _():
            for b in range(K_BLK):
                pltpu.async_copy(rows_v.at[r, b], acc_sh.at[dst_v.at[i3, b]],
                                 sem_s2, add=True)

        @pl.when(jnp.logical_xor(o < CNT_SPLIT, c == 1))
        def _():
            oc = [pltpu.async_copy(ones_v, cnt_sh.at[dst_v.at[i3, b]],
                                   sem_c, add=True)
                  for b in range(K_BLK)]
            for d in oc:
                d.wait()

        @pl.when(jnp.logical_and(o >= 1, r == 1))
        def _():
            for b in range(K_BLK):
                pltpu.make_async_copy(rows_v.at[0, b],
                                      acc_sh.at[dst_v.at[i3, b]],
                                      sem_s).wait()

        @pl.when(jnp.logical_and(o >= 1, r == 0))
        def _():
            for b in range(K_BLK):
                pltpu.make_async_copy(rows_v.at[1, b],
                                      acc_sh.at[dst_v.at[i3, b]],
                                      sem_s2).wait()

        @pl.when(o + 1 < N_BLOCKS)
        def _():
            i3n = lax.rem(o + 1, 3)
            pltpu.make_async_copy(src_hbm.at[blk0 + o + 1],
                                  src_v.at[i3n], sem_i).wait()
            pltpu.make_async_copy(dst_hbm.at[blk0 + o + 1],
                                  dst_v.at[i3n], sem_i).wait()
            xform_idx(i3n)

        @pl.when(o + 2 < N_BLOCKS)
        def _():
            i3nn = lax.rem(o + 2, 3)
            pltpu.async_copy(src_hbm.at[blk0 + o + 2], src_v.at[i3nn], sem_i)
            pltpu.async_copy(dst_hbm.at[blk0 + o + 2], dst_v.at[i3nn], sem_i)

        @pl.when(o + 1 < N_BLOCKS)
        def _():
            fire_gathers(1 - r, lax.rem(o + 1, 3))
        return carry

    lax.fori_loop(0, N_BLOCKS, block, 0)

    _last_r = (N_BLOCKS - 1) % 2
    _last_sem = sem_s2 if _last_r == 1 else sem_s
    for b in range(K_BLK):
        pltpu.make_async_copy(rows_v.at[_last_r, b],
                              acc_sh.at[dst_v.at[0, b]], _last_sem).wait()
    plsc.subcore_barrier()

    pltpu.sync_copy(acc_sh.at[pl.ds(r0, ROWS_PER_TILE)],
                    acc_out.at[pl.ds(r0, ROWS_PER_TILE),
                               pl.ds(c * HALF, HALF)])
    pltpu.sync_copy(cnt_sh.at[pl.ds(r0, ROWS_PER_TILE)],
                    cnt_out.at[pl.ds(r0, ROWS_PER_TILE),
                               pl.ds(c * CNT_W, CNT_W)])


_sc_gather_scatter = functools.partial(
    pl.kernel,
    out_type=(
        jax.ShapeDtypeStruct((N_PAD, D), jnp.float32),
        jax.ShapeDtypeStruct((N_PAD, 2 * CNT_W), jnp.float32),
    ),
    mesh=plsc.VectorSubcoreMesh(core_axis_name="c", subcore_axis_name="s"),
    compiler_params=pltpu.CompilerParams(use_tc_tiling_on_sc=False),
    scratch_types=[
        pltpu.VMEM((3, K_BLK, CHUNK), jnp.int32),
        pltpu.VMEM((3, K_BLK, CHUNK), jnp.int32),
        pltpu.VMEM((2, K_BLK, CHUNK, HALF), jnp.float32),
        pltpu.VMEM((CHUNK, CNT_W), jnp.float32),
        pltpu.VMEM((CHUNK, CNT_W), jnp.float32),
        pltpu.VMEM_SHARED((N_PAD, HALF), jnp.float32),
        pltpu.VMEM_SHARED((N_PAD, CNT_W), jnp.float32),
        pltpu.SemaphoreType.DMA,
        pltpu.SemaphoreType.DMA,
        pltpu.SemaphoreType.DMA,
        pltpu.SemaphoreType.DMA,
        pltpu.SemaphoreType.DMA,
    ],
)(_sc_body)


def _dense_body(acc_ref, cnt_ref, h_ref, wn_ref, ws_ref, b_ref,
                gamma_ref, beta_ref, out_ref):
    summed = acc_ref[:N_NODES, :]
    count = cnt_ref[:N_NODES, 0:1] + cnt_ref[:N_NODES, CNT_W:CNT_W + 1]
    count = jnp.maximum(count, 1.0)
    mean_agg = summed / count
    out = (jnp.dot(mean_agg, wn_ref[...], preferred_element_type=jnp.float32)
           + jnp.dot(h_ref[...], ws_ref[...], preferred_element_type=jnp.float32)
           + b_ref[...])
    out = jnp.maximum(out, 0.0)
    mu = jnp.mean(out, axis=0, keepdims=True)
    var = jnp.mean((out - mu) ** 2, axis=0, keepdims=True)
    out = (out - mu) * lax.rsqrt(var + 1e-5)
    out_ref[...] = out * gamma_ref[...] + beta_ref[...]


_dense = pl.pallas_call(
    _dense_body,
    out_shape=jax.ShapeDtypeStruct((N_NODES, D), jnp.float32),
)


def kernel(feature, edge_index, W_neigh, W_self, b, gamma, beta):
    src = edge_index[0].astype(jnp.int32).reshape(NS * N_BLOCKS, K_BLK, CHUNK)
    dst = edge_index[1].astype(jnp.int32).reshape(NS * N_BLOCKS, K_BLK, CHUNK)
    h2 = feature.reshape(2 * N_NODES, HALF)
    ones = jnp.ones((CHUNK, CNT_W), jnp.float32)
    acc, cnt = _sc_gather_scatter(h2, src, dst, ones)
    return _dense(acc, cnt, feature, W_neigh.T, W_self.T,
                  b.reshape(1, D), gamma.reshape(1, D), beta.reshape(1, D))
